# Initial kernel scaffold; baseline (speedup 1.0000x reference)
import functools

import jax
import jax.numpy as jnp
from jax.experimental import pallas as pl

K = 50
NUM_BINS = 5
CUTOFF = 10.0
BR = 200


def _knn_kernel(m, npad, pr_ref, pat_ref, idx_ref, attr_ref):
    b = pl.program_id(0)
    pr = pr_ref[...]
    pat = pat_ref[...]
    sq_all = jnp.sum(pat * pat, axis=0)
    sq_r = jnp.sum(pr * pr, axis=1)
    g = jnp.dot(pr, pat, preferred_element_type=jnp.float32)
    d2 = sq_r[:, None] + sq_all[None, :] - 2.0 * g
    col = jax.lax.broadcasted_iota(jnp.int32, (BR, npad), 1)
    gi = b * BR + jax.lax.broadcasted_iota(jnp.int32, (BR, npad), 0)
    d2 = jnp.where(col == gi, d2 + 1e10, d2)
    d2 = jnp.where(col >= m, jnp.float32(1e30), d2)

    centers = jnp.linspace(0.0, CUTOFF, NUM_BINS)
    sigma = centers[1] - centers[0]
    inv2s2 = 1.0 / (2.0 * sigma * sigma)

    def body(k, vals):
        mn = jnp.min(vals, axis=1)
        arg = jnp.min(jnp.where(vals == mn[:, None], col, npad), axis=1)
        idx_ref[:, pl.ds(k, 1)] = arg[:, None]
        dist = jnp.sqrt(jnp.maximum(mn, 0.0) + 1e-12)
        a = jnp.exp(-((dist[:, None] - centers[None, :]) ** 2) * inv2s2)
        attr_ref[:, pl.ds(k, 1), :] = a[:, None, :]
        return jnp.where(col == arg[:, None], jnp.float32(1e30), vals)

    jax.lax.fori_loop(0, K, body, d2)


def kernel(pos, keep_idx):
    p = pos[keep_idx]
    M = p.shape[0]
    npad = ((M + 127) // 128) * 128
    pa = jnp.pad(p, ((0, npad - M), (0, 0)))
    pat = pa.T
    grid = (M // BR,)
    nbr, attr = pl.pallas_call(
        functools.partial(_knn_kernel, M, npad),
        grid=grid,
        in_specs=[
            pl.BlockSpec((BR, 3), lambda b: (b, 0)),
            pl.BlockSpec((3, npad), lambda b: (0, 0)),
        ],
        out_specs=[
            pl.BlockSpec((BR, K), lambda b: (b, 0)),
            pl.BlockSpec((BR, K, NUM_BINS), lambda b: (b, 0, 0)),
        ],
        out_shape=[
            jax.ShapeDtypeStruct((M, K), jnp.int32),
            jax.ShapeDtypeStruct((M, K, NUM_BINS), jnp.float32),
        ],
    )(p, pat)

    src = nbr.reshape(-1)
    dst = jnp.repeat(jnp.arange(M, dtype=jnp.int32), K)
    edge_index = jnp.stack([jnp.concatenate([src, dst]),
                            jnp.concatenate([dst, src])])
    A = attr.reshape(-1, NUM_BINS)
    edge_attr = jnp.concatenate([A, A], axis=0)
    return p, edge_index, edge_attr


# speedup vs baseline: 4.9804x; 4.9804x over previous
import functools

import jax
import jax.numpy as jnp
from jax.experimental import pallas as pl

K = 50
NUM_BINS = 5
CUTOFF = 10.0
BR = 200


def _knn_kernel(m, npad, pr_ref, pat_ref, idx_ref, attr_ref):
    b = pl.program_id(0)
    pr = pr_ref[...]
    pat = pat_ref[...]
    sq_all = jnp.sum(pat * pat, axis=0)
    sq_r = jnp.sum(pr * pr, axis=1)
    g = jnp.dot(pr, pat, preferred_element_type=jnp.float32)
    d2 = sq_r[:, None] + sq_all[None, :] - 2.0 * g
    col = jax.lax.broadcasted_iota(jnp.int32, (BR, npad), 1)
    gi = b * BR + jax.lax.broadcasted_iota(jnp.int32, (BR, npad), 0)
    d2 = jnp.where(col == gi, d2 + 1e10, d2)
    d2 = jnp.where(col >= m, jnp.float32(1e30), d2)

    ki = jax.lax.broadcasted_iota(jnp.int32, (BR, K), 1)
    topi0 = jnp.zeros((BR, K), jnp.int32)
    topd0 = jnp.zeros((BR, K), jnp.float32)

    def body(k, carry):
        vals, topi, topd = carry
        mn = jnp.min(vals, axis=1)
        arg = jnp.min(jnp.where(vals == mn[:, None], col, npad), axis=1)
        topi = jnp.where(ki == k, arg[:, None], topi)
        topd = jnp.where(ki == k, mn[:, None], topd)
        vals = jnp.where(col == arg[:, None], jnp.float32(1e30), vals)
        return vals, topi, topd

    _, topi, topd = jax.lax.fori_loop(0, K, body, (d2, topi0, topd0))
    idx_ref[...] = topi
    dist = jnp.sqrt(jnp.maximum(topd, 0.0) + 1e-12)
    centers = jax.lax.broadcasted_iota(
        jnp.int32, (1, 1, NUM_BINS), 2).astype(jnp.float32) * 2.5
    two_s2 = jnp.float32(12.5)
    attr_ref[...] = jnp.exp(-((dist[:, :, None] - centers) ** 2) / two_s2)


def kernel(pos, keep_idx):
    p = pos[keep_idx]
    M = p.shape[0]
    npad = ((M + 127) // 128) * 128
    pa = jnp.pad(p, ((0, npad - M), (0, 0)))
    pat = pa.T
    grid = (M // BR,)
    nbr, attr = pl.pallas_call(
        functools.partial(_knn_kernel, M, npad),
        grid=grid,
        in_specs=[
            pl.BlockSpec((BR, 3), lambda b: (b, 0)),
            pl.BlockSpec((3, npad), lambda b: (0, 0)),
        ],
        out_specs=[
            pl.BlockSpec((BR, K), lambda b: (b, 0)),
            pl.BlockSpec((BR, K, NUM_BINS), lambda b: (b, 0, 0)),
        ],
        out_shape=[
            jax.ShapeDtypeStruct((M, K), jnp.int32),
            jax.ShapeDtypeStruct((M, K, NUM_BINS), jnp.float32),
        ],
    )(p, pat)

    src = nbr.reshape(-1)
    dst = jnp.repeat(jnp.arange(M, dtype=jnp.int32), K)
    edge_index = jnp.stack([jnp.concatenate([src, dst]),
                            jnp.concatenate([dst, src])])
    A = attr.reshape(-1, NUM_BINS)
    edge_attr = jnp.concatenate([A, A], axis=0)
    return p, edge_index, edge_attr
